# baseline (speedup 1.0000x reference)
import functools

import jax
import jax.numpy as jnp
from jax import lax
from jax.experimental import pallas as pl
from jax.experimental.pallas import tpu as pltpu
from jax.experimental.pallas import tpu_sc as plsc

N = 16384
K = 16
D = 128
M = 2
CKD = 64
HID = 128
E = N * K
NW = 32
EPW = E // NW
CH = 128
NCH = EPW // CH

TE = 2048
TR = TE // 128
ER = E // 128
H2 = 2 * HID


def _sc_gather(table, xi, yi):
    mesh = plsc.VectorSubcoreMesh(core_axis_name="c", subcore_axis_name="s")

    @functools.partial(
        pl.kernel,
        out_type=(jax.ShapeDtypeStruct((E, D), jnp.float32),
                  jax.ShapeDtypeStruct((E, D), jnp.float32)),
        mesh=mesh,
        scratch_types=[
            pltpu.VMEM((CH,), jnp.int32),
            pltpu.VMEM((CH,), jnp.int32),
            pltpu.VMEM((CH, D), jnp.float32),
            pltpu.VMEM((CH, D), jnp.float32),
            pltpu.SemaphoreType.DMA,
            pltpu.SemaphoreType.DMA,
        ],
    )
    def k(table_hbm, xi_hbm, yi_hbm, xg_hbm, yg_hbm,
          xidx_v, yidx_v, xrows_v, yrows_v, semx, semy):
        wid = lax.axis_index("s") * 2 + lax.axis_index("c")
        base = wid * EPW

        def body(i, carry):
            off = base + i * CH
            pltpu.sync_copy(xi_hbm.at[pl.ds(off, CH)], xidx_v)
            pltpu.sync_copy(yi_hbm.at[pl.ds(off, CH)], yidx_v)
            cx = pltpu.async_copy(table_hbm.at[xidx_v], xrows_v, semx)
            cy = pltpu.async_copy(table_hbm.at[yidx_v], yrows_v, semy)
            cx.wait()
            pltpu.sync_copy(xrows_v, xg_hbm.at[pl.ds(off, CH)])
            cy.wait()
            pltpu.sync_copy(yrows_v, yg_hbm.at[pl.ds(off, CH)])
            return carry

        lax.fori_loop(0, NCH, body, 0)

    return k(table, xi, yi)


def _act(h):
    he = jnp.maximum(h[:, :HID], 0.0)
    hb = jax.nn.gelu(h[:, HID:])
    return jnp.concatenate([he, hb], axis=1)


def _tc_body(xg_ref, yg_ref, ga0_ref, ga1_ref, ge_ref,
             wx_ref, wy_ref, b0_ref, w1_ref, b1_ref, w2_ref, b2_ref,
             w3_ref, b3_ref, a0_ref, a1_ref, obj_ref):
    f32 = jnp.float32
    bf16 = jnp.bfloat16
    h = jnp.dot(xg_ref[...].astype(bf16), wx_ref[...],
                preferred_element_type=f32)
    h = h + jnp.dot(yg_ref[...].astype(bf16), wy_ref[...],
                    preferred_element_type=f32)
    h = _act(h + b0_ref[...]).astype(bf16)
    h = _act(jnp.dot(h, w1_ref[...], preferred_element_type=f32)
             + b1_ref[...]).astype(bf16)
    h = _act(jnp.dot(h, w2_ref[...], preferred_element_type=f32)
             + b2_ref[...]).astype(bf16)
    o = jnp.dot(h, w3_ref[...], preferred_element_type=f32) + b3_ref[...]
    ec = o[:, :CKD]
    bias = o[:, CKD:CKD + 1]
    ge = ge_ref[...]
    gnorm = jnp.maximum(jnp.sqrt(jnp.sum(ge * ge, axis=1, keepdims=True)), 1e-12)
    gen = ge / gnorm
    ecn = jnp.maximum(jnp.sqrt(jnp.sum(ec * ec, axis=1, keepdims=True)), 1e-12)
    d0 = jnp.sum(ec * gen[0:1, :], axis=1, keepdims=True) / ecn
    d1 = jnp.sum(ec * gen[1:2, :], axis=1, keepdims=True) / ecn
    a0 = jax.nn.sigmoid(d0)[:, 0].reshape(TR, 128)
    a1 = jax.nn.sigmoid(d1)[:, 0].reshape(TR, 128)
    bias = bias[:, 0].reshape(TR, 128)
    a0_ref[...] = a0
    a1_ref[...] = a1
    obj_ref[...] = a0 * (ga0_ref[...] - bias) + a1 * (ga1_ref[...] - bias)


def _tc_mlp(xg, yg, ga0, ga1, ge, wx, wy, b0, w1, b1, w2, b2, w3, b3):
    grid = (E // TE,)
    edge_spec = pl.BlockSpec((TE, D), lambda i: (i, 0))
    col_spec = pl.BlockSpec((TR, 128), lambda i: (i, 0))

    def full(a):
        return pl.BlockSpec(a.shape, lambda i: tuple(0 for _ in a.shape))

    return pl.pallas_call(
        _tc_body,
        grid=grid,
        in_specs=[edge_spec, edge_spec, col_spec, col_spec, full(ge),
                  full(wx), full(wy), full(b0), full(w1), full(b1),
                  full(w2), full(b2), full(w3), full(b3)],
        out_specs=[col_spec, col_spec, col_spec],
        out_shape=[jax.ShapeDtypeStruct((ER, 128), jnp.float32)] * 3,
        compiler_params=pltpu.CompilerParams(
            dimension_semantics=("arbitrary",)),
    )(xg, yg, ga0, ga1, ge, wx, wy, b0, w1, b1, w2, b2, w3, b3)


def kernel(backbone_features, indices, gather_affinities, embeddings,
           enc_W0, enc_b0, enc_W1, enc_b1, enc_W2, enc_b2, enc_W3, enc_b3,
           bp_W0, bp_b0, bp_W1, bp_b1, bp_W2, bp_b2, bp_W3, bp_b3):
    bf = backbone_features[0]
    xi = indices[0, 0].reshape(E)
    yi = indices[0, 1].reshape(E)
    xg, yg = _sc_gather(bf, xi, yi)

    ga = gather_affinities[0].reshape(M, ER, 128)
    ga0 = ga[0]
    ga1 = ga[1]
    ge = jnp.zeros((8, CKD), jnp.float32).at[:M].set(embeddings[:M])

    bf16 = jnp.bfloat16
    z = jnp.zeros((HID, HID), jnp.float32)
    wx = jnp.concatenate([enc_W0[:D], bp_W0[:D]], axis=1).astype(bf16)
    wy = jnp.concatenate([enc_W0[D:], bp_W0[D:]], axis=1).astype(bf16)
    b0 = jnp.concatenate([enc_b0, bp_b0]).reshape(1, H2)
    w1 = jnp.concatenate([jnp.concatenate([enc_W1, z], axis=1),
                          jnp.concatenate([z, bp_W1], axis=1)],
                         axis=0).astype(bf16)
    b1 = jnp.concatenate([enc_b1, bp_b1]).reshape(1, H2)
    w2 = jnp.concatenate([jnp.concatenate([enc_W2, z], axis=1),
                          jnp.concatenate([z, bp_W2], axis=1)],
                         axis=0).astype(bf16)
    b2 = jnp.concatenate([enc_b2, bp_b2]).reshape(1, H2)
    w3_top = jnp.concatenate([enc_W3, jnp.zeros((HID, D - CKD), jnp.float32)],
                             axis=1)
    w3_bot = jnp.zeros((HID, D), jnp.float32).at[:, CKD:CKD + 1].set(bp_W3)
    w3 = jnp.concatenate([w3_top, w3_bot], axis=0).astype(bf16)
    b3 = jnp.zeros((D,), jnp.float32).at[:CKD].set(enc_b3)
    b3 = b3.at[CKD].set(bp_b3[0]).reshape(1, D)

    a0, a1, obj = _tc_mlp(xg, yg, ga0, ga1, ge,
                          wx, wy, b0, w1, b1, w2, b2, w3, b3)

    attn = jnp.stack([a0.reshape(N, K), a1.reshape(N, K)])[None]
    return attn, obj.reshape(1, N, K)


# speedup vs baseline: 2.0369x; 2.0369x over previous
import functools

import jax
import jax.numpy as jnp
from jax import lax
from jax.experimental import pallas as pl
from jax.experimental.pallas import tpu as pltpu
from jax.experimental.pallas import tpu_sc as plsc

N = 16384
K = 16
D = 128
M = 2
CKD = 64
HID = 128
E = N * K
NW = 32
EPW = E // NW
CH = 128
NCH = EPW // CH

TE = 2048
TR = TE // 128
ER = E // 128
H2 = 2 * HID


def _sc_gather(table, xi, yi):
    mesh = plsc.VectorSubcoreMesh(core_axis_name="c", subcore_axis_name="s")

    @functools.partial(
        pl.kernel,
        out_type=(jax.ShapeDtypeStruct((E, D), jnp.float32),
                  jax.ShapeDtypeStruct((E, D), jnp.float32)),
        mesh=mesh,
        scratch_types=[
            pltpu.VMEM((CH,), jnp.int32),
            pltpu.VMEM((CH,), jnp.int32),
            pltpu.VMEM((CH, D), jnp.float32),
            pltpu.VMEM((CH, D), jnp.float32),
            pltpu.SemaphoreType.DMA,
            pltpu.SemaphoreType.DMA,
        ],
    )
    def k(table_hbm, xi_hbm, yi_hbm, xg_hbm, yg_hbm,
          xidx_v, yidx_v, xrows_v, yrows_v, semx, semy):
        wid = lax.axis_index("s") * 2 + lax.axis_index("c")
        base = wid * EPW

        def body(i, carry):
            off = base + i * CH
            pltpu.sync_copy(xi_hbm.at[pl.ds(off, CH)], xidx_v)
            pltpu.sync_copy(yi_hbm.at[pl.ds(off, CH)], yidx_v)
            cx = pltpu.async_copy(table_hbm.at[xidx_v], xrows_v, semx)
            cy = pltpu.async_copy(table_hbm.at[yidx_v], yrows_v, semy)
            cx.wait()
            pltpu.sync_copy(xrows_v, xg_hbm.at[pl.ds(off, CH)])
            cy.wait()
            pltpu.sync_copy(yrows_v, yg_hbm.at[pl.ds(off, CH)])
            return carry

        lax.fori_loop(0, NCH, body, 0)

    return k(table, xi, yi)


_CT = (((1,), (1,)), ((), ()))


def _act(h):
    h = h.astype(jnp.bfloat16)
    he = jnp.maximum(h[:HID], jnp.bfloat16(0))
    hb = jax.nn.gelu(h[HID:])
    return jnp.concatenate([he, hb], axis=0)


def _tc_body(xg_ref, yg_ref, ga0_ref, ga1_ref, ge_ref,
             wx_ref, wy_ref, b0_ref, w1_ref, b1_ref, w2_ref, b2_ref,
             w3_ref, b3_ref, a0_ref, a1_ref, obj_ref):
    f32 = jnp.float32
    bf16 = jnp.bfloat16
    h = jax.lax.dot_general(wx_ref[...], xg_ref[...].astype(bf16), _CT,
                            preferred_element_type=f32)
    h = h + jax.lax.dot_general(wy_ref[...], yg_ref[...].astype(bf16), _CT,
                                preferred_element_type=f32)
    h = _act(h + b0_ref[...])
    h = _act(jnp.dot(w1_ref[...], h, preferred_element_type=f32)
             + b1_ref[...])
    h = _act(jnp.dot(w2_ref[...], h, preferred_element_type=f32)
             + b2_ref[...])
    o = jnp.dot(w3_ref[...], h, preferred_element_type=f32) + b3_ref[...]
    ec = o[:CKD]
    ge = ge_ref[...]
    gnorm = jnp.maximum(jnp.sqrt(jnp.sum(ge * ge, axis=1, keepdims=True)), 1e-12)
    gen = (ge / gnorm).astype(bf16)
    ind = (jax.lax.broadcasted_iota(jnp.int32, (8, CKD), 0) == 2).astype(bf16)
    vt = jnp.concatenate([gen, ind], axis=1)
    ecb = ec.astype(bf16)
    pt = jnp.concatenate([ecb, ecb * ecb], axis=0)
    rt = jnp.dot(vt, pt, preferred_element_type=f32)
    d0 = rt[0].reshape(TR, 128)
    d1 = rt[1].reshape(TR, 128)
    s = rt[2].reshape(TR, 128)
    bias = o[CKD].reshape(TR, 128)
    ecn = jnp.maximum(jnp.sqrt(s), 1e-12)
    a0 = jax.nn.sigmoid(d0 / ecn)
    a1 = jax.nn.sigmoid(d1 / ecn)
    a0_ref[...] = a0
    a1_ref[...] = a1
    obj_ref[...] = a0 * (ga0_ref[...] - bias) + a1 * (ga1_ref[...] - bias)


def _tc_mlp(xg, yg, ga0, ga1, ge, wx, wy, b0, w1, b1, w2, b2, w3, b3):
    grid = (E // TE,)
    edge_spec = pl.BlockSpec((TE, D), lambda i: (i, 0))
    col_spec = pl.BlockSpec((TR, 128), lambda i: (i, 0))

    def full(a):
        return pl.BlockSpec(a.shape, lambda i: tuple(0 for _ in a.shape))

    return pl.pallas_call(
        _tc_body,
        grid=grid,
        in_specs=[edge_spec, edge_spec, col_spec, col_spec, full(ge),
                  full(wx), full(wy), full(b0), full(w1), full(b1),
                  full(w2), full(b2), full(w3), full(b3)],
        out_specs=[col_spec, col_spec, col_spec],
        out_shape=[jax.ShapeDtypeStruct((ER, 128), jnp.float32)] * 3,
        compiler_params=pltpu.CompilerParams(
            dimension_semantics=("arbitrary",)),
    )(xg, yg, ga0, ga1, ge, wx, wy, b0, w1, b1, w2, b2, w3, b3)


def kernel(backbone_features, indices, gather_affinities, embeddings,
           enc_W0, enc_b0, enc_W1, enc_b1, enc_W2, enc_b2, enc_W3, enc_b3,
           bp_W0, bp_b0, bp_W1, bp_b1, bp_W2, bp_b2, bp_W3, bp_b3):
    bf = backbone_features[0]
    xi = indices[0, 0].reshape(E)
    yi = indices[0, 1].reshape(E)
    xg, yg = _sc_gather(bf, xi, yi)

    ga = gather_affinities[0].reshape(M, ER, 128)
    ga0 = ga[0]
    ga1 = ga[1]
    ge = jnp.zeros((8, CKD), jnp.float32).at[:M].set(embeddings[:M])

    bf16 = jnp.bfloat16
    z = jnp.zeros((HID, HID), jnp.float32)
    wx = jnp.concatenate([enc_W0[:D], bp_W0[:D]], axis=1).T.astype(bf16)
    wy = jnp.concatenate([enc_W0[D:], bp_W0[D:]], axis=1).T.astype(bf16)
    b0 = jnp.concatenate([enc_b0, bp_b0]).reshape(H2, 1)
    w1 = jnp.concatenate([jnp.concatenate([enc_W1, z], axis=1),
                          jnp.concatenate([z, bp_W1], axis=1)],
                         axis=0).T.astype(bf16)
    b1 = jnp.concatenate([enc_b1, bp_b1]).reshape(H2, 1)
    w2 = jnp.concatenate([jnp.concatenate([enc_W2, z], axis=1),
                          jnp.concatenate([z, bp_W2], axis=1)],
                         axis=0).T.astype(bf16)
    b2 = jnp.concatenate([enc_b2, bp_b2]).reshape(H2, 1)
    w3_top = jnp.concatenate([enc_W3, jnp.zeros((HID, D - CKD), jnp.float32)],
                             axis=1)
    w3_bot = jnp.zeros((HID, D), jnp.float32).at[:, CKD:CKD + 1].set(bp_W3)
    w3 = jnp.concatenate([w3_top, w3_bot], axis=0).T.astype(bf16)
    b3 = jnp.zeros((D,), jnp.float32).at[:CKD].set(enc_b3)
    b3 = b3.at[CKD].set(bp_b3[0]).reshape(D, 1)

    a0, a1, obj = _tc_mlp(xg, yg, ga0, ga1, ge,
                          wx, wy, b0, w1, b1, w2, b2, w3, b3)

    attn = jnp.stack([a0.reshape(N, K), a1.reshape(N, K)])[None]
    return attn, obj.reshape(1, N, K)


# speedup vs baseline: 2.3632x; 1.1602x over previous
import functools

import jax
import jax.numpy as jnp
from jax import lax
from jax.experimental import pallas as pl
from jax.experimental.pallas import tpu as pltpu
from jax.experimental.pallas import tpu_sc as plsc

N = 16384
K = 16
D = 128
M = 2
CKD = 64
HID = 128
E = N * K
NW = 32
EPW = E // NW
CH = 128
NCH = EPW // CH

TE = 2048
TR = TE // 128
ER = E // 128
H2 = 2 * HID


def _sc_gather(table, xi, yi):
    mesh = plsc.VectorSubcoreMesh(core_axis_name="c", subcore_axis_name="s")

    @functools.partial(
        pl.kernel,
        out_type=(jax.ShapeDtypeStruct((E, D), jnp.float32),
                  jax.ShapeDtypeStruct((E, D), jnp.float32)),
        mesh=mesh,
        scratch_types=[
            pltpu.VMEM((NCH, CH), jnp.int32),
            pltpu.VMEM((NCH, CH), jnp.int32),
            pltpu.VMEM((CH, D), jnp.float32),
            pltpu.VMEM((CH, D), jnp.float32),
            pltpu.VMEM((CH, D), jnp.float32),
            pltpu.VMEM((CH, D), jnp.float32),
            pltpu.SemaphoreType.DMA,
            pltpu.SemaphoreType.DMA,
            pltpu.SemaphoreType.DMA,
            pltpu.SemaphoreType.DMA,
        ],
    )
    def k(table_hbm, xi_hbm, yi_hbm, xg_hbm, yg_hbm,
          xidx_v, yidx_v, xb0, xb1, yb0, yb1, sx0, sx1, sy0, sy1):
        wid = lax.axis_index("s") * 2 + lax.axis_index("c")
        base = wid * EPW
        xbufs, ybufs = (xb0, xb1), (yb0, yb1)
        sxs, sys = (sx0, sx1), (sy0, sy1)

        pltpu.sync_copy(xi_hbm.at[pl.ds(wid * NCH, NCH)], xidx_v)
        pltpu.sync_copy(yi_hbm.at[pl.ds(wid * NCH, NCH)], yidx_v)

        def issue(i, b):
            pltpu.async_copy(table_hbm.at[xidx_v.at[i]], xbufs[b], sxs[b])
            pltpu.async_copy(table_hbm.at[yidx_v.at[i]], ybufs[b], sys[b])

        def drain(i, b):
            pltpu.make_async_copy(table_hbm.at[xidx_v.at[i]],
                                  xbufs[b], sxs[b]).wait()
            off = base + i * CH
            pltpu.sync_copy(xbufs[b], xg_hbm.at[pl.ds(off, CH)])
            pltpu.make_async_copy(table_hbm.at[yidx_v.at[i]],
                                  ybufs[b], sys[b]).wait()
            pltpu.sync_copy(ybufs[b], yg_hbm.at[pl.ds(off, CH)])

        issue(0, 0)

        def body(jj, carry):
            for b in (0, 1):
                i = 2 * jj + b

                @pl.when(i + 1 < NCH)
                def _():
                    issue(i + 1, 1 - b)

                drain(i, b)
            return carry

        lax.fori_loop(0, NCH // 2, body, 0)

    return k(table, xi, yi)


_CT = (((1,), (1,)), ((), ()))


def _act(h):
    h = h.astype(jnp.bfloat16)
    he = jnp.maximum(h[:HID], jnp.bfloat16(0))
    hb = jax.nn.gelu(h[HID:])
    return jnp.concatenate([he, hb], axis=0)


def _tc_body(xg_ref, yg_ref, ga0_ref, ga1_ref, ge_ref,
             wx_ref, wy_ref, b0_ref, w1_ref, b1_ref, w2_ref, b2_ref,
             w3_ref, b3_ref, a0_ref, a1_ref, obj_ref):
    f32 = jnp.float32
    bf16 = jnp.bfloat16
    h = jax.lax.dot_general(wx_ref[...], xg_ref[...].astype(bf16), _CT,
                            preferred_element_type=f32)
    h = h + jax.lax.dot_general(wy_ref[...], yg_ref[...].astype(bf16), _CT,
                                preferred_element_type=f32)
    h = _act(h + b0_ref[...])
    h = _act(jnp.dot(w1_ref[...], h, preferred_element_type=f32)
             + b1_ref[...])
    h = _act(jnp.dot(w2_ref[...], h, preferred_element_type=f32)
             + b2_ref[...])
    o = jnp.dot(w3_ref[...], h, preferred_element_type=f32) + b3_ref[...]
    ec = o[:CKD]
    ge = ge_ref[...]
    gnorm = jnp.maximum(jnp.sqrt(jnp.sum(ge * ge, axis=1, keepdims=True)), 1e-12)
    gen = (ge / gnorm).astype(bf16)
    ind = (jax.lax.broadcasted_iota(jnp.int32, (8, CKD), 0) == 2).astype(bf16)
    vt = jnp.concatenate([gen, ind], axis=1)
    ecb = ec.astype(bf16)
    pt = jnp.concatenate([ecb, ecb * ecb], axis=0)
    rt = jnp.dot(vt, pt, preferred_element_type=f32)
    d0 = rt[0].reshape(TR, 128)
    d1 = rt[1].reshape(TR, 128)
    s = rt[2].reshape(TR, 128)
    bias = o[CKD].reshape(TR, 128)
    ecn = jnp.maximum(jnp.sqrt(s), 1e-12)
    a0 = jax.nn.sigmoid(d0 / ecn)
    a1 = jax.nn.sigmoid(d1 / ecn)
    a0_ref[...] = a0
    a1_ref[...] = a1
    obj_ref[...] = a0 * (ga0_ref[...] - bias) + a1 * (ga1_ref[...] - bias)


def _tc_mlp(xg, yg, ga0, ga1, ge, wx, wy, b0, w1, b1, w2, b2, w3, b3):
    grid = (E // TE,)
    edge_spec = pl.BlockSpec((TE, D), lambda i: (i, 0))
    col_spec = pl.BlockSpec((TR, 128), lambda i: (i, 0))

    def full(a):
        return pl.BlockSpec(a.shape, lambda i: tuple(0 for _ in a.shape))

    return pl.pallas_call(
        _tc_body,
        grid=grid,
        in_specs=[edge_spec, edge_spec, col_spec, col_spec, full(ge),
                  full(wx), full(wy), full(b0), full(w1), full(b1),
                  full(w2), full(b2), full(w3), full(b3)],
        out_specs=[col_spec, col_spec, col_spec],
        out_shape=[jax.ShapeDtypeStruct((ER, 128), jnp.float32)] * 3,
        compiler_params=pltpu.CompilerParams(
            dimension_semantics=("arbitrary",)),
    )(xg, yg, ga0, ga1, ge, wx, wy, b0, w1, b1, w2, b2, w3, b3)


def kernel(backbone_features, indices, gather_affinities, embeddings,
           enc_W0, enc_b0, enc_W1, enc_b1, enc_W2, enc_b2, enc_W3, enc_b3,
           bp_W0, bp_b0, bp_W1, bp_b1, bp_W2, bp_b2, bp_W3, bp_b3):
    bf = backbone_features[0]
    xi = indices[0, 0].reshape(NW * NCH, CH)
    yi = indices[0, 1].reshape(NW * NCH, CH)
    xg, yg = _sc_gather(bf, xi, yi)

    ga = gather_affinities[0].reshape(M, ER, 128)
    ga0 = ga[0]
    ga1 = ga[1]
    ge = jnp.zeros((8, CKD), jnp.float32).at[:M].set(embeddings[:M])

    bf16 = jnp.bfloat16
    z = jnp.zeros((HID, HID), jnp.float32)
    wx = jnp.concatenate([enc_W0[:D], bp_W0[:D]], axis=1).T.astype(bf16)
    wy = jnp.concatenate([enc_W0[D:], bp_W0[D:]], axis=1).T.astype(bf16)
    b0 = jnp.concatenate([enc_b0, bp_b0]).reshape(H2, 1)
    w1 = jnp.concatenate([jnp.concatenate([enc_W1, z], axis=1),
                          jnp.concatenate([z, bp_W1], axis=1)],
                         axis=0).T.astype(bf16)
    b1 = jnp.concatenate([enc_b1, bp_b1]).reshape(H2, 1)
    w2 = jnp.concatenate([jnp.concatenate([enc_W2, z], axis=1),
                          jnp.concatenate([z, bp_W2], axis=1)],
                         axis=0).T.astype(bf16)
    b2 = jnp.concatenate([enc_b2, bp_b2]).reshape(H2, 1)
    w3_top = jnp.concatenate([enc_W3, jnp.zeros((HID, D - CKD), jnp.float32)],
                             axis=1)
    w3_bot = jnp.zeros((HID, D), jnp.float32).at[:, CKD:CKD + 1].set(bp_W3)
    w3 = jnp.concatenate([w3_top, w3_bot], axis=0).T.astype(bf16)
    b3 = jnp.zeros((D,), jnp.float32).at[:CKD].set(enc_b3)
    b3 = b3.at[CKD].set(bp_b3[0]).reshape(D, 1)

    a0, a1, obj = _tc_mlp(xg, yg, ga0, ga1, ge,
                          wx, wy, b0, w1, b1, w2, b2, w3, b3)

    attn = jnp.stack([a0.reshape(N, K), a1.reshape(N, K)])[None]
    return attn, obj.reshape(1, N, K)


# speedup vs baseline: 2.5889x; 1.0955x over previous
import functools

import jax
import jax.numpy as jnp
from jax import lax
from jax.experimental import pallas as pl
from jax.experimental.pallas import tpu as pltpu
from jax.experimental.pallas import tpu_sc as plsc

N = 16384
K = 16
D = 128
M = 2
CKD = 64
HID = 128
E = N * K
NW = 32
EPW = E // NW
CH = 128
NCH = EPW // CH

TE = 2048
TR = TE // 128
ER = E // 128
H2 = 2 * HID


def _sc_gather(table, xi, yi, n):
    epw = n // NW
    nch = epw // CH
    mesh = plsc.VectorSubcoreMesh(core_axis_name="c", subcore_axis_name="s")

    @functools.partial(
        pl.kernel,
        out_type=(jax.ShapeDtypeStruct((n, D), jnp.float32),
                  jax.ShapeDtypeStruct((n, D), jnp.float32)),
        mesh=mesh,
        scratch_types=[
            pltpu.VMEM((nch, CH), jnp.int32),
            pltpu.VMEM((nch, CH), jnp.int32),
            pltpu.VMEM((CH, D), jnp.float32),
            pltpu.VMEM((CH, D), jnp.float32),
            pltpu.VMEM((CH, D), jnp.float32),
            pltpu.VMEM((CH, D), jnp.float32),
            pltpu.SemaphoreType.DMA,
            pltpu.SemaphoreType.DMA,
            pltpu.SemaphoreType.DMA,
            pltpu.SemaphoreType.DMA,
        ],
    )
    def k(table_hbm, xi_hbm, yi_hbm, xg_hbm, yg_hbm,
          xidx_v, yidx_v, xb0, xb1, yb0, yb1, sx0, sx1, sy0, sy1):
        wid = lax.axis_index("s") * 2 + lax.axis_index("c")
        base = wid * epw
        xbufs, ybufs = (xb0, xb1), (yb0, yb1)
        sxs, sys = (sx0, sx1), (sy0, sy1)

        pltpu.sync_copy(xi_hbm.at[pl.ds(wid * nch, nch)], xidx_v)
        pltpu.sync_copy(yi_hbm.at[pl.ds(wid * nch, nch)], yidx_v)

        def issue(i, b):
            pltpu.async_copy(table_hbm.at[xidx_v.at[i]], xbufs[b], sxs[b])
            pltpu.async_copy(table_hbm.at[yidx_v.at[i]], ybufs[b], sys[b])

        def drain(i, b):
            pltpu.make_async_copy(table_hbm.at[xidx_v.at[i]],
                                  xbufs[b], sxs[b]).wait()
            off = base + i * CH
            pltpu.sync_copy(xbufs[b], xg_hbm.at[pl.ds(off, CH)])
            pltpu.make_async_copy(table_hbm.at[yidx_v.at[i]],
                                  ybufs[b], sys[b]).wait()
            pltpu.sync_copy(ybufs[b], yg_hbm.at[pl.ds(off, CH)])

        issue(0, 0)

        def body(jj, carry):
            for b in (0, 1):
                i = 2 * jj + b

                @pl.when(i + 1 < nch)
                def _():
                    issue(i + 1, 1 - b)

                drain(i, b)
            return carry

        lax.fori_loop(0, nch // 2, body, 0)

    return k(table, xi, yi)


_CT = (((1,), (1,)), ((), ()))


def _act(h):
    h = h.astype(jnp.bfloat16)
    he = jnp.maximum(h[:HID], jnp.bfloat16(0))
    hb = jax.nn.gelu(h[HID:])
    return jnp.concatenate([he, hb], axis=0)


def _tc_body(xg_ref, yg_ref, ga0_ref, ga1_ref, ge_ref,
             wx_ref, wy_ref, b0_ref, w1_ref, b1_ref, w2_ref, b2_ref,
             w3_ref, b3_ref, a0_ref, a1_ref, obj_ref):
    f32 = jnp.float32
    bf16 = jnp.bfloat16
    h = jax.lax.dot_general(wx_ref[...], xg_ref[...].astype(bf16), _CT,
                            preferred_element_type=f32)
    h = h + jax.lax.dot_general(wy_ref[...], yg_ref[...].astype(bf16), _CT,
                                preferred_element_type=f32)
    h = _act(h + b0_ref[...])
    h = _act(jnp.dot(w1_ref[...], h, preferred_element_type=f32)
             + b1_ref[...])
    h = _act(jnp.dot(w2_ref[...], h, preferred_element_type=f32)
             + b2_ref[...])
    o = jnp.dot(w3_ref[...], h, preferred_element_type=f32) + b3_ref[...]
    ec = o[:CKD]
    ge = ge_ref[...]
    gnorm = jnp.maximum(jnp.sqrt(jnp.sum(ge * ge, axis=1, keepdims=True)), 1e-12)
    gen = (ge / gnorm).astype(bf16)
    ind = (jax.lax.broadcasted_iota(jnp.int32, (8, CKD), 0) == 2).astype(bf16)
    vt = jnp.concatenate([gen, ind], axis=1)
    ecb = ec.astype(bf16)
    pt = jnp.concatenate([ecb, ecb * ecb], axis=0)
    rt = jnp.dot(vt, pt, preferred_element_type=f32)
    d0 = rt[0].reshape(TR, 128)
    d1 = rt[1].reshape(TR, 128)
    s = rt[2].reshape(TR, 128)
    bias = o[CKD].reshape(TR, 128)
    ecn = jnp.maximum(jnp.sqrt(s), 1e-12)
    a0 = jax.nn.sigmoid(d0 / ecn)
    a1 = jax.nn.sigmoid(d1 / ecn)
    a0_ref[...] = a0
    a1_ref[...] = a1
    obj_ref[...] = a0 * (ga0_ref[...] - bias) + a1 * (ga1_ref[...] - bias)


def _tc_mlp(xg, yg, ga0, ga1, ge, wx, wy, b0, w1, b1, w2, b2, w3, b3):
    n = xg.shape[0]
    grid = (n // TE,)
    edge_spec = pl.BlockSpec((TE, D), lambda i: (i, 0))
    col_spec = pl.BlockSpec((TR, 128), lambda i: (i, 0))

    def full(a):
        return pl.BlockSpec(a.shape, lambda i: tuple(0 for _ in a.shape))

    return pl.pallas_call(
        _tc_body,
        grid=grid,
        in_specs=[edge_spec, edge_spec, col_spec, col_spec, full(ge),
                  full(wx), full(wy), full(b0), full(w1), full(b1),
                  full(w2), full(b2), full(w3), full(b3)],
        out_specs=[col_spec, col_spec, col_spec],
        out_shape=[jax.ShapeDtypeStruct((n // 128, 128), jnp.float32)] * 3,
        compiler_params=pltpu.CompilerParams(
            dimension_semantics=("arbitrary",)),
    )(xg, yg, ga0, ga1, ge, wx, wy, b0, w1, b1, w2, b2, w3, b3)


def kernel(backbone_features, indices, gather_affinities, embeddings,
           enc_W0, enc_b0, enc_W1, enc_b1, enc_W2, enc_b2, enc_W3, enc_b3,
           bp_W0, bp_b0, bp_W1, bp_b1, bp_W2, bp_b2, bp_W3, bp_b3):
    bf = backbone_features[0]
    xi = indices[0, 0].reshape(NW * NCH, CH)
    yi = indices[0, 1].reshape(NW * NCH, CH)

    ga = gather_affinities[0].reshape(M, ER, 128)
    ga0 = ga[0]
    ga1 = ga[1]
    ge = jnp.zeros((8, CKD), jnp.float32).at[:M].set(embeddings[:M])

    bf16 = jnp.bfloat16
    z = jnp.zeros((HID, HID), jnp.float32)
    wx = jnp.concatenate([enc_W0[:D], bp_W0[:D]], axis=1).T.astype(bf16)
    wy = jnp.concatenate([enc_W0[D:], bp_W0[D:]], axis=1).T.astype(bf16)
    b0 = jnp.concatenate([enc_b0, bp_b0]).reshape(H2, 1)
    w1 = jnp.concatenate([jnp.concatenate([enc_W1, z], axis=1),
                          jnp.concatenate([z, bp_W1], axis=1)],
                         axis=0).T.astype(bf16)
    b1 = jnp.concatenate([enc_b1, bp_b1]).reshape(H2, 1)
    w2 = jnp.concatenate([jnp.concatenate([enc_W2, z], axis=1),
                          jnp.concatenate([z, bp_W2], axis=1)],
                         axis=0).T.astype(bf16)
    b2 = jnp.concatenate([enc_b2, bp_b2]).reshape(H2, 1)
    w3_top = jnp.concatenate([enc_W3, jnp.zeros((HID, D - CKD), jnp.float32)],
                             axis=1)
    w3_bot = jnp.zeros((HID, D), jnp.float32).at[:, CKD:CKD + 1].set(bp_W3)
    w3 = jnp.concatenate([w3_top, w3_bot], axis=0).T.astype(bf16)
    b3 = jnp.zeros((D,), jnp.float32).at[:CKD].set(enc_b3)
    b3 = b3.at[CKD].set(bp_b3[0]).reshape(D, 1)

    NSEG = 2
    rows = (NW * NCH) // NSEG
    crows = ER // NSEG
    parts = []
    for s in range(NSEG):
        xg_s, yg_s = _sc_gather(bf, xi[s * rows:(s + 1) * rows],
                                yi[s * rows:(s + 1) * rows], E // NSEG)
        parts.append((xg_s, yg_s,
                      ga0[s * crows:(s + 1) * crows],
                      ga1[s * crows:(s + 1) * crows]))
    outs = [_tc_mlp(xg_s, yg_s, g0, g1, ge,
                    wx, wy, b0, w1, b1, w2, b2, w3, b3)
            for xg_s, yg_s, g0, g1 in parts]
    a0 = jnp.concatenate([o[0] for o in outs])
    a1 = jnp.concatenate([o[1] for o in outs])
    obj = jnp.concatenate([o[2] for o in outs])

    attn = jnp.stack([a0.reshape(N, K), a1.reshape(N, K)])[None]
    return attn, obj.reshape(1, N, K)


# speedup vs baseline: 2.6342x; 1.0175x over previous
import functools

import jax
import jax.numpy as jnp
from jax import lax
from jax.experimental import pallas as pl
from jax.experimental.pallas import tpu as pltpu
from jax.experimental.pallas import tpu_sc as plsc

N = 16384
K = 16
D = 128
M = 2
CKD = 64
HID = 128
E = N * K
NW = 32
EPW = E // NW
CH = 128
NCH = EPW // CH

TE = 2048
TR = TE // 128
ER = E // 128
H2 = 2 * HID


def _sc_gather(table, xi, yi, n):
    epw = n // NW
    nch = epw // CH
    mesh = plsc.VectorSubcoreMesh(core_axis_name="c", subcore_axis_name="s")

    @functools.partial(
        pl.kernel,
        out_type=(jax.ShapeDtypeStruct((n, D), jnp.float32),
                  jax.ShapeDtypeStruct((n, D), jnp.float32)),
        mesh=mesh,
        scratch_types=[
            pltpu.VMEM((nch, CH), jnp.int32),
            pltpu.VMEM((nch, CH), jnp.int32),
            pltpu.VMEM((CH, D), jnp.float32),
            pltpu.VMEM((CH, D), jnp.float32),
            pltpu.VMEM((CH, D), jnp.float32),
            pltpu.VMEM((CH, D), jnp.float32),
            pltpu.SemaphoreType.DMA,
            pltpu.SemaphoreType.DMA,
            pltpu.SemaphoreType.DMA,
            pltpu.SemaphoreType.DMA,
        ],
    )
    def k(table_hbm, xi_hbm, yi_hbm, xg_hbm, yg_hbm,
          xidx_v, yidx_v, xb0, xb1, yb0, yb1, sx0, sx1, sy0, sy1):
        wid = lax.axis_index("s") * 2 + lax.axis_index("c")
        base = wid * epw
        xbufs, ybufs = (xb0, xb1), (yb0, yb1)
        sxs, sys = (sx0, sx1), (sy0, sy1)

        pltpu.sync_copy(xi_hbm.at[pl.ds(wid * nch, nch)], xidx_v)
        pltpu.sync_copy(yi_hbm.at[pl.ds(wid * nch, nch)], yidx_v)

        def issue(i, b):
            pltpu.async_copy(table_hbm.at[xidx_v.at[i]], xbufs[b], sxs[b])
            pltpu.async_copy(table_hbm.at[yidx_v.at[i]], ybufs[b], sys[b])

        def drain(i, b):
            pltpu.make_async_copy(table_hbm.at[xidx_v.at[i]],
                                  xbufs[b], sxs[b]).wait()
            off = base + i * CH
            pltpu.sync_copy(xbufs[b], xg_hbm.at[pl.ds(off, CH)])
            pltpu.make_async_copy(table_hbm.at[yidx_v.at[i]],
                                  ybufs[b], sys[b]).wait()
            pltpu.sync_copy(ybufs[b], yg_hbm.at[pl.ds(off, CH)])

        issue(0, 0)

        def body(jj, carry):
            for b in (0, 1):
                i = 2 * jj + b

                @pl.when(i + 1 < nch)
                def _():
                    issue(i + 1, 1 - b)

                drain(i, b)
            return carry

        lax.fori_loop(0, nch // 2, body, 0)

    return k(table, xi, yi)


_CT = (((1,), (1,)), ((), ()))


def _act(h):
    h = h.astype(jnp.bfloat16)
    he = jnp.maximum(h[:HID], jnp.bfloat16(0))
    hb = jax.nn.gelu(h[HID:])
    return jnp.concatenate([he, hb], axis=0)


def _tc_body(xg_ref, yg_ref, ga0_ref, ga1_ref, ge_ref,
             wx_ref, wy_ref, b0_ref, w1_ref, b1_ref, w2_ref, b2_ref,
             w3_ref, b3_ref, a0_ref, a1_ref, obj_ref):
    f32 = jnp.float32
    bf16 = jnp.bfloat16
    h = jax.lax.dot_general(wx_ref[...], xg_ref[...].astype(bf16), _CT,
                            preferred_element_type=f32)
    h = h + jax.lax.dot_general(wy_ref[...], yg_ref[...].astype(bf16), _CT,
                                preferred_element_type=f32)
    h = _act(h + b0_ref[...])
    h = _act(jnp.dot(w1_ref[...], h, preferred_element_type=f32)
             + b1_ref[...])
    h = _act(jnp.dot(w2_ref[...], h, preferred_element_type=f32)
             + b2_ref[...])
    o = jnp.dot(w3_ref[...], h, preferred_element_type=f32) + b3_ref[...]
    ec = o[:CKD]
    ge = ge_ref[...]
    gnorm = jnp.maximum(jnp.sqrt(jnp.sum(ge * ge, axis=1, keepdims=True)), 1e-12)
    gen = (ge / gnorm).astype(bf16)
    ind = (jax.lax.broadcasted_iota(jnp.int32, (8, CKD), 0) == 2).astype(bf16)
    vt = jnp.concatenate([gen, ind], axis=1)
    ecb = ec.astype(bf16)
    pt = jnp.concatenate([ecb, ecb * ecb], axis=0)
    rt = jnp.dot(vt, pt, preferred_element_type=f32)
    d0 = rt[0].reshape(TR, 128)
    d1 = rt[1].reshape(TR, 128)
    s = rt[2].reshape(TR, 128)
    bias = o[CKD].reshape(TR, 128)
    ecn = jnp.maximum(jnp.sqrt(s), 1e-12)
    a0 = jax.nn.sigmoid(d0 / ecn)
    a1 = jax.nn.sigmoid(d1 / ecn)
    a0_ref[...] = a0
    a1_ref[...] = a1
    obj_ref[...] = a0 * (ga0_ref[...] - bias) + a1 * (ga1_ref[...] - bias)


def _tc_mlp(xg, yg, ga0, ga1, ge, wx, wy, b0, w1, b1, w2, b2, w3, b3):
    n = xg.shape[0]
    grid = (n // TE,)
    edge_spec = pl.BlockSpec((TE, D), lambda i: (i, 0))
    col_spec = pl.BlockSpec((TR, 128), lambda i: (i, 0))

    def full(a):
        return pl.BlockSpec(a.shape, lambda i: tuple(0 for _ in a.shape))

    return pl.pallas_call(
        _tc_body,
        grid=grid,
        in_specs=[edge_spec, edge_spec, col_spec, col_spec, full(ge),
                  full(wx), full(wy), full(b0), full(w1), full(b1),
                  full(w2), full(b2), full(w3), full(b3)],
        out_specs=[col_spec, col_spec, col_spec],
        out_shape=[jax.ShapeDtypeStruct((n // 128, 128), jnp.float32)] * 3,
        compiler_params=pltpu.CompilerParams(
            dimension_semantics=("arbitrary",)),
    )(xg, yg, ga0, ga1, ge, wx, wy, b0, w1, b1, w2, b2, w3, b3)


def kernel(backbone_features, indices, gather_affinities, embeddings,
           enc_W0, enc_b0, enc_W1, enc_b1, enc_W2, enc_b2, enc_W3, enc_b3,
           bp_W0, bp_b0, bp_W1, bp_b1, bp_W2, bp_b2, bp_W3, bp_b3):
    bf = backbone_features[0]
    xi = indices[0, 0].reshape(NW * NCH, CH)
    yi = indices[0, 1].reshape(NW * NCH, CH)

    ga = gather_affinities[0].reshape(M, ER, 128)
    ga0 = ga[0]
    ga1 = ga[1]
    ge = jnp.zeros((8, CKD), jnp.float32).at[:M].set(embeddings[:M])

    bf16 = jnp.bfloat16
    z = jnp.zeros((HID, HID), jnp.float32)
    wx = jnp.concatenate([enc_W0[:D], bp_W0[:D]], axis=1).T.astype(bf16)
    wy = jnp.concatenate([enc_W0[D:], bp_W0[D:]], axis=1).T.astype(bf16)
    b0 = jnp.concatenate([enc_b0, bp_b0]).reshape(H2, 1)
    w1 = jnp.concatenate([jnp.concatenate([enc_W1, z], axis=1),
                          jnp.concatenate([z, bp_W1], axis=1)],
                         axis=0).T.astype(bf16)
    b1 = jnp.concatenate([enc_b1, bp_b1]).reshape(H2, 1)
    w2 = jnp.concatenate([jnp.concatenate([enc_W2, z], axis=1),
                          jnp.concatenate([z, bp_W2], axis=1)],
                         axis=0).T.astype(bf16)
    b2 = jnp.concatenate([enc_b2, bp_b2]).reshape(H2, 1)
    w3_top = jnp.concatenate([enc_W3, jnp.zeros((HID, D - CKD), jnp.float32)],
                             axis=1)
    w3_bot = jnp.zeros((HID, D), jnp.float32).at[:, CKD:CKD + 1].set(bp_W3)
    w3 = jnp.concatenate([w3_top, w3_bot], axis=0).T.astype(bf16)
    b3 = jnp.zeros((D,), jnp.float32).at[:CKD].set(enc_b3)
    b3 = b3.at[CKD].set(bp_b3[0]).reshape(D, 1)

    NSEG = 4
    rows = (NW * NCH) // NSEG
    crows = ER // NSEG
    parts = []
    for s in range(NSEG):
        xg_s, yg_s = _sc_gather(bf, xi[s * rows:(s + 1) * rows],
                                yi[s * rows:(s + 1) * rows], E // NSEG)
        parts.append((xg_s, yg_s,
                      ga0[s * crows:(s + 1) * crows],
                      ga1[s * crows:(s + 1) * crows]))
    outs = [_tc_mlp(xg_s, yg_s, g0, g1, ge,
                    wx, wy, b0, w1, b1, w2, b2, w3, b3)
            for xg_s, yg_s, g0, g1 in parts]
    a0 = jnp.concatenate([o[0] for o in outs])
    a1 = jnp.concatenate([o[1] for o in outs])
    obj = jnp.concatenate([o[2] for o in outs])

    attn = jnp.stack([a0.reshape(N, K), a1.reshape(N, K)])[None]
    return attn, obj.reshape(1, N, K)
